# baseline (speedup 1.0000x reference)
import functools

import jax
import jax.numpy as jnp
from jax import lax
from jax.experimental import pallas as pl
from jax.experimental.pallas import tpu as pltpu
from jax.experimental.pallas import tpu_sc as plsc

N = 10000
E = 320000
IN_DIM = 128
H = 256
L = 4
NC = 10

NCORE = 2
NSUB = 16
NBUCK = 32
N_PAD = 10240
EB = 128
NBLK = 86
BK = NBLK * EB
E_PAD = NBUCK * BK
NBLKP = NBLK + 2
BKP = NBLKP * EB
E_PAD2 = NBUCK * BKP
ROWS_B = N_PAD // NBUCK
AROWS = 328

_INTERPRET = False
RBLK = 256
RBLK_E = 512


def _embed_body(h_ref, w_ref, b_ref, o_ref):
    acc = jnp.dot(h_ref[...], w_ref[...], preferred_element_type=jnp.float32)
    acc = acc + b_ref[0:1, :]
    for c in range(4):
        o_ref[c] = acc[:, c * 64:(c + 1) * 64]


def _embed(h_pad, Wh, bh_t):
    R = RBLK
    return pl.pallas_call(
        _embed_body,
        grid=(N_PAD // R,),
        in_specs=[
            pl.BlockSpec((R, IN_DIM), lambda i: (i, 0)),
            pl.BlockSpec((IN_DIM, H), lambda i: (0, 0)),
            pl.BlockSpec((8, H), lambda i: (0, 0)),
        ],
        out_specs=pl.BlockSpec((4, R, 64), lambda i: (0, i, 0)),
        out_shape=jax.ShapeDtypeStruct((4, N_PAD, 64), jnp.float32),
        interpret=_INTERPRET,
    )(h_pad, Wh, bh_t)


def _tables_body(hh_ref, w_ref, b_ref, a_ref, ts_ref, td_ref):
    R = hh_ref.shape[1]
    acc = jnp.zeros((R, 4 * H), jnp.float32)
    for c in range(4):
        acc = acc + jnp.dot(hh_ref[c], w_ref[pl.ds(c * 64, 64), :],
                            preferred_element_type=jnp.float32)
    acc = acc + b_ref[0:1, :]
    for c in range(4):
        a_ref[c] = acc[:, c * 64:(c + 1) * 64]
        d_c = acc[:, 512 + c * 64:512 + (c + 1) * 64]
        b_c = acc[:, 256 + c * 64:256 + (c + 1) * 64]
        ts_ref[c] = jnp.concatenate([d_c, b_c], axis=1)
        td_ref[c] = acc[:, 768 + c * 64:768 + (c + 1) * 64]


def _tables(hh, Wcat, bcat_t):
    R = RBLK
    return pl.pallas_call(
        _tables_body,
        grid=(N_PAD // R,),
        in_specs=[
            pl.BlockSpec((4, R, 64), lambda i: (0, i, 0)),
            pl.BlockSpec((H, 4 * H), lambda i: (0, 0)),
            pl.BlockSpec((8, 4 * H), lambda i: (0, 0)),
        ],
        out_specs=[
            pl.BlockSpec((4, R, 64), lambda i: (0, i, 0)),
            pl.BlockSpec((4, R, 128), lambda i: (0, i, 0)),
            pl.BlockSpec((4, R, 64), lambda i: (0, i, 0)),
        ],
        out_shape=[
            jax.ShapeDtypeStruct((4, N_PAD, 64), jnp.float32),
            jax.ShapeDtypeStruct((4, N_PAD, 128), jnp.float32),
            jax.ShapeDtypeStruct((4, N_PAD, 64), jnp.float32),
        ],
        interpret=_INTERPRET,
    )(hh, Wcat, bcat_t)


def _ce_body(nr, *refs):
    r_refs = refs[:nr]
    w_refs = refs[nr:2 * nr]
    o_ref = refs[2 * nr]
    R = r_refs[0].shape[1]
    acc = jnp.zeros((R, H), jnp.float32)
    for j in range(nr):
        for c in range(4):
            acc = acc + jnp.dot(r_refs[j][c], w_refs[j][pl.ds(c * 64, 64), :],
                                preferred_element_type=jnp.float32)
    for c in range(4):
        o_ref[c] = acc[:, c * 64:(c + 1) * 64]


def _ce_matmul(rs, Wfs):
    nr = len(rs)
    R = RBLK_E
    return pl.pallas_call(
        functools.partial(_ce_body, nr),
        grid=(E_PAD // R,),
        in_specs=[pl.BlockSpec((4, R, 64), lambda i: (0, i, 0)) for _ in range(nr)]
        + [pl.BlockSpec((H, H), lambda i: (0, 0)) for _ in range(nr)],
        out_specs=pl.BlockSpec((4, R, 64), lambda i: (0, i, 0)),
        out_shape=jax.ShapeDtypeStruct((4, E_PAD, 64), jnp.float32),
        interpret=_INTERPRET,
    )(*rs, *Wfs)


def _hupd_body(a_ref, nd_ref, hh_ref, g_ref, b_ref, o_ref):
    a = a_ref[0]
    ndv = nd_ref[0]
    hhv = hh_ref[0]
    num = ndv[:, :64]
    den = ndv[:, 64:] + 1e-6
    t = jnp.maximum(a + num / den, 0.0)
    tv = t[:N, :]
    s1 = jnp.sum(tv, axis=0)
    s2 = jnp.sum(tv * tv, axis=0)
    m = s1 / N
    v = s2 / N - m * m
    scale = g_ref[0, 0] * lax.rsqrt(v + 1e-5)
    shift = b_ref[0, 0] - scale * m
    o_ref[0] = hhv + t * scale[None, :] + shift[None, :]


def _hupd(A, nd, hh, g4, b4):
    return pl.pallas_call(
        _hupd_body,
        grid=(4,),
        in_specs=[
            pl.BlockSpec((1, N_PAD, 64), lambda c: (c, 0, 0)),
            pl.BlockSpec((1, N_PAD, 128), lambda c: (c, 0, 0)),
            pl.BlockSpec((1, N_PAD, 64), lambda c: (c, 0, 0)),
            pl.BlockSpec((1, 1, 64), lambda c: (c, 0, 0)),
            pl.BlockSpec((1, 1, 64), lambda c: (c, 0, 0)),
        ],
        out_specs=pl.BlockSpec((1, N_PAD, 64), lambda c: (c, 0, 0)),
        out_shape=jax.ShapeDtypeStruct((4, N_PAD, 64), jnp.float32),
        interpret=_INTERPRET,
    )(A, nd, hh, g4, b4)


def _final_body(hh_ref, w_ref, b_ref, o_ref):
    R = hh_ref.shape[1]
    acc = jnp.zeros((R, 128), jnp.float32)
    for c in range(4):
        acc = acc + jnp.dot(hh_ref[c], w_ref[pl.ds(c * 64, 64), :],
                            preferred_element_type=jnp.float32)
    o_ref[...] = acc + b_ref[0:1, :]


def _final(hh, Woutp, bout_t):
    R = RBLK
    return pl.pallas_call(
        _final_body,
        grid=(N_PAD // R,),
        in_specs=[
            pl.BlockSpec((4, R, 64), lambda i: (0, i, 0)),
            pl.BlockSpec((H, 128), lambda i: (0, 0)),
            pl.BlockSpec((8, 128), lambda i: (0, 0)),
        ],
        out_specs=pl.BlockSpec((R, 128), lambda i: (i, 0)),
        out_shape=jax.ShapeDtypeStruct((N_PAD, 128), jnp.float32),
        interpret=_INTERPRET,
    )(hh, Woutp, bout_t)


def _sc_edge_call(has_ce, has_r, ce_flat, tsrc, tdst, idxs, uc, zrow):
    mesh = plsc.VectorSubcoreMesh(core_axis_name="c", subcore_axis_name="s",
                                  num_cores=NCORE, num_subcores=NSUB)
    out_type = []
    if has_r:
        out_type.append(jax.ShapeDtypeStruct((4 * E_PAD, 64), jnp.float32))
    out_type.append(jax.ShapeDtypeStruct((4, N_PAD * 128), jnp.float32))
    if has_r:
        out_type.append(jax.ShapeDtypeStruct((128, 2, 64), jnp.float32))

    scratch = (
        [pltpu.VMEM((4, EB), jnp.float32)] * 2
        + [pltpu.VMEM((EB,), jnp.int32)] * 4
        + [pltpu.VMEM((EB, 128), jnp.float32)] * 2
        + [pltpu.VMEM((EB, 64), jnp.float32)] * 2
        + [pltpu.VMEM((EB, 64), jnp.float32)] * 2
        + [pltpu.VMEM((EB, 64), jnp.float32)]
        + [pltpu.VMEM((2, 64), jnp.float32)] * 2
        + [pltpu.VMEM((AROWS * 128,), jnp.float32)]
        + [pltpu.SemaphoreType.DMA] * 3
    )

    def body(*refs):
        ins = list(refs)
        if has_ce:
            ce_r = ins.pop(0)
        tsrc_r, tdst_r, idxs_r, uc_r, zrow_r = ins[:5]
        ins = ins[5:]
        if has_r:
            r_out, nd_out, bn_out = ins[:3]
            ins = ins[3:]
        else:
            nd_out = ins.pop(0)
        (ib0, ib1, sa0, sa1, da0, da1, tg0, tg1, td0, td1,
         cb0, cb1, rb, ucb, bnv, acc, sgt, sgd, sgc) = ins
        idxb = [ib0, ib1]
        sab = [sa0, sa1]
        dab = [da0, da1]
        tsg = [tg0, tg1]
        tdg = [td0, td1]
        ceb = [cb0, cb1]

        cid = lax.axis_index("c")
        sid = lax.axis_index("s")
        w = cid * NSUB + sid
        base_e = w * BK
        ones16 = jnp.ones((16,), jnp.float32)
        ones16i = jnp.ones((16,), jnp.int32)
        iota16 = lax.iota(jnp.int32, 16)
        ba = [iota16 + j * 16 for j in range(4)]
        ba2 = [iota16 + 64 + j * 16 for j in range(4)]

        def load_build(b, p, qn):
            pltpu.sync_copy(idxs_r.at[w, b], idxb[p])
            for v in range(EB // 16):
                sl = pl.ds(v * 16, 16)
                sab[p][sl] = idxb[p][0, sl].astype(jnp.int32) + qn
                dab[p][sl] = idxb[p][1, sl].astype(jnp.int32) + qn

        def issue_gathers(b, p, qe):
            hs = [pltpu.async_copy(tsrc_r.at[sab[p]], tsg[p], sgt),
                  pltpu.async_copy(tdst_r.at[dab[p]], tdg[p], sgd)]
            if has_ce:
                bce = jnp.minimum(b, NBLK - 1)
                hs.append(pltpu.async_copy(ce_r.at[pl.ds(qe + bce * EB, EB)],
                                           ceb[p], sgc))
            return hs

        def sweep_body(q, _unused):
            qn = q * N_PAD
            qe = q * E_PAD + base_e
            pltpu.sync_copy(uc_r.at[q], ucb)
            uvec = [ucb[0, pl.ds(j * 16, 16)] for j in range(4)]
            cvec = [ucb[1, pl.ds(j * 16, 16)] for j in range(4)]
            pltpu.sync_copy(zrow_r, acc)

            def compute(b, p, carry):
                tsgp, tdgp, cebp, rbp, idxp = (
                    tsg[p], tdg[p], ceb[p], rb, idxb[p])

                def grp_body(g, c2):
                    bs = list(c2[:4])
                    bq = list(c2[4:])
                    gsl = pl.ds(g * 16, 16)
                    dl16 = idxp[2, gsl]
                    er16 = idxp[3, gsl]
                    gof = g * 16
                    for k2 in range(16):
                        k = gof + k2
                        dlf = dl16[k2]
                        rowv = (dlf.astype(jnp.int32) * 128) * ones16i
                        eru = er16[k2] * ones16
                        if has_r:
                            bv16 = jnp.where(dlf < ROWS_B, 1.0, 0.0) * ones16
                        for j in range(4):
                            sl = pl.ds(j * 16, 16)
                            sl2 = pl.ds(64 + j * 16, 16)
                            x = (tsgp[k, sl] + tdgp[k, sl] + eru * uvec[j]
                                 + cvec[j])
                            if has_ce:
                                x = x + cebp[k, sl]
                            sg = 1.0 / (1.0 + jnp.exp(-x))
                            plsc.addupdate_scatter(acc, [rowv + ba[j]],
                                                   sg * tsgp[k, sl2])
                            plsc.addupdate_scatter(acc, [rowv + ba2[j]], sg)
                            if has_r:
                                r = jnp.maximum(x, 0.0)
                                rbp[k, sl] = r
                                rv = r * bv16
                                bs[j] = bs[j] + rv
                                bq[j] = bq[j] + rv * r
                    return (*bs, *bq)

                return lax.fori_loop(0, EB // 16, grp_body, carry)

            def write_r(b, p):
                if has_r:
                    pltpu.sync_copy(rb, r_out.at[pl.ds(qe + b * EB, EB)])

            load_build(0, 0, qn)
            h0 = issue_gathers(0, 0, qe)
            load_build(1, 1, qn)
            for h in h0:
                h.wait()

            def pair_body(jj, carry):
                for pp in range(2):
                    i = 2 * jj + 1 + pp
                    p = (1 + pp) % 2
                    hs = issue_gathers(i, p, qe)
                    carry = compute(i - 1, 1 - p, carry)
                    write_r(i - 1, 1 - p)
                    load_build(i + 1, 1 - p, qn)
                    for h in hs:
                        h.wait()
                return carry

            z = jnp.zeros((16,), jnp.float32)
            carry = lax.fori_loop(0, (NBLKP - 2) // 2, pair_body, (z,) * 8)

            pltpu.sync_copy(acc.at[pl.ds(0, ROWS_B * 128)],
                            nd_out.at[q, pl.ds(w * ROWS_B * 128,
                                               ROWS_B * 128)])
            if has_r:
                for j in range(4):
                    sl = pl.ds(j * 16, 16)
                    bnv[0, sl] = carry[j]
                    bnv[1, sl] = carry[4 + j]
                pltpu.sync_copy(bnv, bn_out.at[q * NBUCK + w])
            return 0

        lax.fori_loop(0, 4, sweep_body, 0)

    ins = []
    if has_ce:
        ins.append(ce_flat)
    ins += [tsrc, tdst, idxs, uc, zrow]
    k = pl.kernel(body, out_type=out_type, mesh=mesh, scratch_types=scratch,
                  compiler_params=pltpu.CompilerParams(
                      use_tc_tiling_on_sc=False,
                      needs_layout_passes=False),
                  interpret=_INTERPRET)
    outs = k(*ins)
    if has_r:
        return outs[0], outs[1], outs[2]
    return None, outs[0], None


def _tile8(v):
    return jnp.tile(v[None, :], (8, 1))


def kernel(h, edge_index, e, Wh, bh, We, be, layW, layb, gh, bh_bn, ge, be_bn,
           Wout, bout):
    f32 = jnp.float32
    h_pad = jnp.zeros((N_PAD, IN_DIM), f32).at[:N].set(h)
    src = edge_index[0]
    dst = edge_index[1]
    bucket = dst // ROWS_B
    order = jnp.argsort(bucket, stable=True)
    src_s = src[order]
    dst_s = dst[order]
    er_s = e[:, 0][order]
    bkt_s = bucket[order]
    starts = jnp.searchsorted(bkt_s, jnp.arange(NBUCK, dtype=bkt_s.dtype))
    dest = bkt_s * BKP + (jnp.arange(E) - starts[bkt_s])
    src_p = jnp.full((E_PAD2,), N, jnp.int32).at[dest].set(src_s)
    dg_p = jnp.full((E_PAD2,), N, jnp.int32).at[dest].set(dst_s)
    dl_p = jnp.full((E_PAD2,), AROWS - 1, jnp.int32).at[dest].set(
        dst_s - bkt_s * ROWS_B)
    er_p = jnp.zeros((E_PAD2,), f32).at[dest].set(er_s)
    idx_pack = jnp.stack(
        [v.astype(f32).reshape(NBUCK, NBLKP, EB)
         for v in (src_p, dg_p, dl_p)]
        + [er_p.reshape(NBUCK, NBLKP, EB)], axis=2)
    zrow = jnp.zeros((AROWS * 128,), f32)

    hh = _embed(h_pad, Wh, _tile8(bh))

    rs = []
    scales = []
    shift_sum = jnp.zeros((H,), f32)

    for i in range(L):
        Wcat = jnp.concatenate([layW[i, 0], layW[i, 1], layW[i, 3],
                                layW[i, 4]], axis=1)
        bcat = jnp.concatenate([layb[i, 0], layb[i, 1], layb[i, 3],
                                layb[i, 4]])
        A, tsrc, tdst = _tables(hh, Wcat, _tile8(bcat))

        Wi = layW[i, 2]
        u = (We @ Wi)[0]
        const = (be + shift_sum) @ Wi + layb[i, 2]
        uc = jnp.stack([u.reshape(4, 64), const.reshape(4, 64)], axis=1)

        if i > 0:
            Wfs = [s[:, None] * Wi for s in scales]
            ce = _ce_matmul(rs, Wfs).reshape(4 * E_PAD, 64)
        else:
            ce = None

        has_r = i < L - 1
        r_i, nd, bn = _sc_edge_call(
            i > 0, has_r, ce,
            tsrc.reshape(4 * N_PAD, 128), tdst.reshape(4 * N_PAD, 64),
            idx_pack, uc, zrow)

        hh = _hupd(A, nd.reshape(4, N_PAD, 128), hh, gh[i].reshape(4, 1, 64),
                   bh_bn[i].reshape(4, 1, 64))

        if has_r:
            bnr = bn.reshape(4, NBUCK, 2, 64).sum(axis=1)
            s1 = bnr[:, 0, :].reshape(H)
            s2 = bnr[:, 1, :].reshape(H)
            m = s1 / E
            v = s2 / E - m * m
            sc = ge[i] / jnp.sqrt(v + 1e-5)
            sh = be_bn[i] - sc * m
            rs.append(r_i.reshape(4, E_PAD, 64))
            scales.append(sc)
            shift_sum = shift_sum + sh

    Woutp = jnp.zeros((H, 128), f32).at[:, :NC].set(Wout)
    out = _final(hh, Woutp, _tile8(jnp.zeros((128,), f32).at[:NC].set(bout)))
    return out[:N, :NC]
